# baseline (speedup 1.0000x reference)
import functools

import jax
import jax.numpy as jnp
from jax import lax
from jax.experimental import pallas as pl
from jax.experimental.pallas import tpu as pltpu
from jax.experimental.pallas import tpu_sc as plsc

GAMMA_ = 0.05
LOG2_ = 0.6931471805599453

_NC, _NS, _LANES = 2, 16, 16
_NW = _NC * _NS

_B, _S, _K, _LY = 4096, 500, 200, 50
_PADQ = 100001
_MASKN = 100096

_RPW = _B // _NW
_CH = 8
_NCHUNK = _RPW // _CH


def _tc_sp_body(vals_ref, sp_ref):
    v = vals_ref[...]
    sp_ref[...] = jnp.maximum(v, 0.0) - v + jnp.log(1.0 + jnp.exp(-jnp.abs(v)))


def _tc_sp(vals):
    blk = 512
    return pl.pallas_call(
        _tc_sp_body,
        grid=(_B // blk,),
        in_specs=[pl.BlockSpec((blk, _K), lambda i: (i, 0))],
        out_specs=pl.BlockSpec((blk, _K), lambda i: (i, 0)),
        out_shape=jax.ShapeDtypeStruct((_B, _K), jnp.float32),
    )(vals)


def _tc_a_body(out_ref, a_ref):
    step = pl.program_id(0)
    x = out_ref[...]
    a_part = jnp.sum(jnp.maximum(x, 0.0) + jnp.log(1.0 + jnp.exp(-jnp.abs(x))))

    @pl.when(step == 0)
    def _():
        a_ref[0, 0] = 0.0

    a_ref[0, 0] += a_part


def _tc_a(out):
    blk = 512
    return pl.pallas_call(
        _tc_a_body,
        grid=(_B // blk,),
        in_specs=[pl.BlockSpec((blk, _S), lambda i: (i, 0))],
        out_specs=pl.BlockSpec(memory_space=pltpu.SMEM),
        out_shape=jax.ShapeDtypeStruct((1, 1), jnp.float32),
    )(out)


def _sc_body(sh_hbm, y_hbm, tk_hbm, o_hbm, sp_hbm,
             res_out,
             mask_v, sh_v0, tk_v0, sp_v0,
             sh_v1, tk_v1, sp_v1,
             y_v, o_v,
             r1_v, r2_v, r3_v, r4_v, sem0, sem1, sem_y, sem_o):
    wid = lax.axis_index("s") * _NC + lax.axis_index("c")
    row0 = wid * _RPW
    neg1 = jnp.full((_LANES,), -1, jnp.int32)
    bufs = ((sh_v0, tk_v0, sp_v0), (sh_v1, tk_v1, sp_v1))
    sems = (sem0, sem1)

    def _copies(ci, slot):
        base = row0 + ci * _CH
        sh_v, tk_v, sp_v = bufs[slot]
        return (
            (sh_hbm.at[pl.ds(base, _CH)], sh_v),
            (tk_hbm.at[pl.ds(base, _CH)], tk_v),
            (sp_hbm.at[pl.ds(base, _CH)], sp_v),
        )

    def _issue(ci, slot):
        for src, dst in _copies(ci, slot):
            pltpu.async_copy(src, dst, sems[slot])

    def _wait(ci, slot):
        for src, dst in _copies(ci, slot):
            pltpu.make_async_copy(src, dst, sems[slot]).wait()

    def _issue_y(ci):
        base = row0 + ci * _CH
        pltpu.async_copy(y_hbm.at[pl.ds(base, _CH)], y_v, sem_y)

    def _wait_y(ci):
        base = row0 + ci * _CH
        pltpu.make_async_copy(y_hbm.at[pl.ds(base, _CH)], y_v, sem_y).wait()

    def _issue_o(ci):
        base = row0 + ci * _CH
        pltpu.async_copy(o_hbm.at[pl.ds(base, _CH)], o_v, sem_o)

    def _wait_o(ci):
        base = row0 + ci * _CH
        pltpu.make_async_copy(o_hbm.at[pl.ds(base, _CH)], o_v, sem_o).wait()

    _issue(0, 0)
    _issue(1, 1)
    _issue_y(0)
    _issue_o(0)

    def init_body(i, _):
        for t in range(16):
            mask_v[pl.ds(i * 256 + t * 16, 16)] = neg1
        return 0

    lax.fori_loop(0, _MASKN // 256, init_body, 0)

    zf = jnp.zeros((_LANES,), jnp.float32)
    zi = jnp.zeros((_LANES,), jnp.int32)
    onei = jnp.full((_LANES,), 1, jnp.int32)
    iota = lax.iota(jnp.int32, _LANES)
    padq = jnp.full((_LANES,), _PADQ, jnp.int32)
    m_sh = iota < (_S % 16)
    m_tk = iota < (_K % 16)
    cy = jnp.minimum(iota + (_LY // 16) * 16, _LY - 1)
    csh = jnp.minimum(iota + (_S // 16) * 16, _S - 1)
    ctk = jnp.minimum(iota + (_K // 16) * 16, _K - 1)
    n_y, n_sh, n_tk = _LY // 16, _S // 16, _K // 16
    last = _NCHUNK - 1

    def _row(r, base, slot, carry, issues):
        s1, h, csum, cmax = carry
        sh_v, tk_v, sp_v = bufs[slot]
        tag = jnp.full((_LANES,), base + r, jnp.int32)
        rowi = jnp.full((_LANES,), r, jnp.int32)
        for t in range(n_y):
            plsc.store_scatter(mask_v, [y_v[r, pl.ds(t * 16, 16)]], tag)
        plsc.store_scatter(mask_v, [plsc.load_gather(y_v, [rowi, cy])], tag)
        if issues is not None:
            _issue_y(issues)
        for j in range(n_sh + 1):
            if j < n_sh:
                q = sh_v[r, pl.ds(j * 16, 16)]
                o = o_v[r, pl.ds(j * 16, 16)]
            else:
                q = jnp.where(m_sh, plsc.load_gather(sh_v, [rowi, csh]),
                              padq)
                o = plsc.load_gather(o_v, [rowi, csh])
            m = plsc.load_gather(mask_v, [q])
            s1 = s1 + jnp.where(m == tag, o, zf)
        if issues is not None:
            _issue_o(issues)
        rowcnt = zi
        for j in range(n_tk + 1):
            if j < n_tk:
                q = tk_v[r, pl.ds(j * 16, 16)]
                spv = sp_v[r, pl.ds(j * 16, 16)]
            else:
                q = jnp.where(m_tk, plsc.load_gather(tk_v, [rowi, ctk]),
                              padq)
                spv = plsc.load_gather(sp_v, [rowi, ctk])
            m = plsc.load_gather(mask_v, [q])
            hit = m == tag
            h = h + jnp.where(hit, spv, zf)
            rowcnt = rowcnt + jnp.where(hit, onei, zi)
        cmax = cmax + 0
        csum = csum + rowcnt
        return (s1, h, csum, cmax)

    def _compute(ci, slot, carry):
        base = row0 + ci * _CH
        carry = lax.fori_loop(
            0, _CH - 1, lambda r, c: _row(r, base, slot, c, None), carry)
        return _row(_CH - 1, base, slot, carry,
                    jnp.minimum(ci + 1, last))

    def pair_body(p, carry):
        c0 = p * 2
        c1 = c0 + 1
        _wait(c0, 0)
        _wait_y(c0)
        _wait_o(c0)
        carry = _compute(c0, 0, carry)
        _issue(jnp.minimum(c0 + 2, last), 0)
        _wait(c1, 1)
        _wait_y(c1)
        _wait_o(c1)
        carry = _compute(c1, 1, carry)
        _issue(jnp.minimum(c1 + 2, last), 1)
        return carry

    s1, h, csum, cmax = lax.fori_loop(
        0, _NCHUNK // 2, pair_body,
        (zf, zf, zi, jnp.int32(0)))
    _wait(last, 0)
    _wait(last, 1)
    _wait_y(last)
    _wait_o(last)

    nwl = _NW * _LANES
    r1_v[...] = s1
    r2_v[...] = h
    r3_v[...] = csum.astype(jnp.float32)
    r4_v[...] = jnp.full((_LANES,), cmax, jnp.int32).astype(jnp.float32)
    pltpu.sync_copy(r1_v, res_out.at[pl.ds(wid * _LANES, _LANES)])
    pltpu.sync_copy(r2_v, res_out.at[pl.ds(nwl + wid * _LANES, _LANES)])
    pltpu.sync_copy(r3_v, res_out.at[pl.ds(2 * nwl + wid * _LANES, _LANES)])
    pltpu.sync_copy(r4_v, res_out.at[pl.ds(3 * nwl + wid * _LANES, _LANES)])


def _sc_membership(sh, y, tk, out, sp):
    mesh = plsc.VectorSubcoreMesh(core_axis_name="c", subcore_axis_name="s")
    buf_pair = [
        pltpu.VMEM((_CH, _S), jnp.int32),
        pltpu.VMEM((_CH, _K), jnp.int32),
        pltpu.VMEM((_CH, _K), jnp.float32),
    ]
    f = pl.kernel(
        _sc_body,
        out_type=jax.ShapeDtypeStruct((4 * _NW * _LANES,), jnp.float32),
        mesh=mesh,
        compiler_params=pltpu.CompilerParams(
            needs_layout_passes=False, use_tc_tiling_on_sc=True),
        scratch_types=(
            [pltpu.VMEM((_MASKN,), jnp.int32)]
            + buf_pair + buf_pair
            + [
                pltpu.VMEM((_CH, _LY), jnp.int32),
                pltpu.VMEM((_CH, _S), jnp.float32),
                pltpu.VMEM((_LANES,), jnp.float32),
                pltpu.VMEM((_LANES,), jnp.float32),
                pltpu.VMEM((_LANES,), jnp.float32),
                pltpu.VMEM((_LANES,), jnp.float32),
                pltpu.SemaphoreType.DMA,
                pltpu.SemaphoreType.DMA,
                pltpu.SemaphoreType.DMA,
                pltpu.SemaphoreType.DMA,
            ]
        ),
    )
    return f(sh, y, tk, out, sp)


def kernel(out, shorty, topk_C_vals, topk_C_inds, y_inds):
    B, S = out.shape
    sp = _tc_sp(topk_C_vals)
    res = _sc_membership(shorty.astype(jnp.int32), y_inds.astype(jnp.int32),
                         topk_C_inds.astype(jnp.int32), out, sp)
    a_arr = _tc_a(out)

    nwl = _NW * _LANES
    A = a_arr[0, 0]
    S1 = jnp.sum(res[:nwl])
    H = jnp.sum(res[nwl:2 * nwl])
    csum = jnp.sum(res[2 * nwl:3 * nwl])
    k = jnp.maximum(jnp.max(res[3 * nwl:]), 1.0)
    loss_precision = (A - S1) / jnp.float32(B * S)
    n = jnp.float32(B) * k
    loss_recall = (H + (n - csum) * jnp.float32(LOG2_)) / n
    return loss_precision + jnp.float32(GAMMA_) * loss_recall


# speedup vs baseline: 1.0419x; 1.0419x over previous
import functools

import jax
import jax.numpy as jnp
from jax import lax
from jax.experimental import pallas as pl
from jax.experimental.pallas import tpu as pltpu
from jax.experimental.pallas import tpu_sc as plsc

GAMMA_ = 0.05
LOG2_ = 0.6931471805599453

_NC, _NS, _LANES = 2, 16, 16
_NW = _NC * _NS

_B, _S, _K, _LY = 4096, 500, 200, 50
_PADQ = 100001
_MASKN = 100096

_RPW = _B // _NW
_CH = 8
_NCHUNK = _RPW // _CH


def _tc_sp_body(vals_ref, sp_ref):
    v = vals_ref[...]
    sp = jnp.maximum(v, 0.0) - v + jnp.log(1.0 + jnp.exp(-jnp.abs(v)))
    sp_ref[...] = sp.T


def _tc_sp(vals_t):
    blk = 512
    return pl.pallas_call(
        _tc_sp_body,
        grid=(_B // blk,),
        in_specs=[pl.BlockSpec((_K, blk), lambda i: (0, i))],
        out_specs=pl.BlockSpec((blk, _K), lambda i: (i, 0)),
        out_shape=jax.ShapeDtypeStruct((_B, _K), jnp.float32),
    )(vals_t)


def _tc_a_body(out_ref, a_ref):
    step = pl.program_id(0)
    x = out_ref[...]
    a_part = jnp.sum(jnp.maximum(x, 0.0) + jnp.log(1.0 + jnp.exp(-jnp.abs(x))))

    @pl.when(step == 0)
    def _():
        a_ref[0, 0] = 0.0

    a_ref[0, 0] += a_part


def _tc_a(out):
    blk = 512
    return pl.pallas_call(
        _tc_a_body,
        grid=(_B // blk,),
        in_specs=[pl.BlockSpec((blk, _S), lambda i: (i, 0))],
        out_specs=pl.BlockSpec(memory_space=pltpu.SMEM),
        out_shape=jax.ShapeDtypeStruct((1, 1), jnp.float32),
    )(out)


def _sc_body(sh_hbm, y_hbm, tk_hbm, o_hbm, sp_hbm,
             res_out,
             mask_v, sh_v0, tk_v0, sp_v0,
             sh_v1, tk_v1, sp_v1,
             y_v, o_v,
             r1_v, r2_v, r3_v, r4_v, sem0, sem1, sem_y, sem_o):
    wid = lax.axis_index("s") * _NC + lax.axis_index("c")
    row0 = wid * _RPW
    neg1 = jnp.full((_LANES,), -1, jnp.int32)
    bufs = ((sh_v0, tk_v0, sp_v0), (sh_v1, tk_v1, sp_v1))
    sems = (sem0, sem1)

    def _copies(ci, slot):
        base = row0 + ci * _CH
        sh_v, tk_v, sp_v = bufs[slot]
        return (
            (sh_hbm.at[pl.ds(base, _CH)], sh_v),
            (tk_hbm.at[pl.ds(base, _CH)], tk_v),
            (sp_hbm.at[pl.ds(base, _CH)], sp_v),
        )

    def _issue(ci, slot):
        for src, dst in _copies(ci, slot):
            pltpu.async_copy(src, dst, sems[slot])

    def _wait(ci, slot):
        for src, dst in _copies(ci, slot):
            pltpu.make_async_copy(src, dst, sems[slot]).wait()

    def _issue_y(ci):
        base = row0 + ci * _CH
        pltpu.async_copy(y_hbm.at[pl.ds(base, _CH)], y_v, sem_y)

    def _wait_y(ci):
        base = row0 + ci * _CH
        pltpu.make_async_copy(y_hbm.at[pl.ds(base, _CH)], y_v, sem_y).wait()

    def _issue_o(ci):
        base = row0 + ci * _CH
        pltpu.async_copy(o_hbm.at[pl.ds(base, _CH)], o_v, sem_o)

    def _wait_o(ci):
        base = row0 + ci * _CH
        pltpu.make_async_copy(o_hbm.at[pl.ds(base, _CH)], o_v, sem_o).wait()

    _issue(0, 0)
    _issue(1, 1)
    _issue_y(0)
    _issue_o(0)

    def init_body(i, _):
        for t in range(16):
            mask_v[pl.ds(i * 256 + t * 16, 16)] = neg1
        return 0

    lax.fori_loop(0, _MASKN // 256, init_body, 0)

    zf = jnp.zeros((_LANES,), jnp.float32)
    zi = jnp.zeros((_LANES,), jnp.int32)
    onei = jnp.full((_LANES,), 1, jnp.int32)
    iota = lax.iota(jnp.int32, _LANES)
    padq = jnp.full((_LANES,), _PADQ, jnp.int32)
    m_sh = iota < (_S % 16)
    m_tk = iota < (_K % 16)
    cy = jnp.minimum(iota + (_LY // 16) * 16, _LY - 1)
    csh = jnp.minimum(iota + (_S // 16) * 16, _S - 1)
    ctk = jnp.minimum(iota + (_K // 16) * 16, _K - 1)
    n_y, n_sh, n_tk = _LY // 16, _S // 16, _K // 16
    last = _NCHUNK - 1

    def _row(r, base, slot, carry, issues):
        s1, h, csum, cmax = carry
        sh_v, tk_v, sp_v = bufs[slot]
        tag = jnp.full((_LANES,), base + r, jnp.int32)
        rowi = jnp.full((_LANES,), r, jnp.int32)
        for t in range(n_y):
            plsc.store_scatter(mask_v, [y_v[r, pl.ds(t * 16, 16)]], tag)
        plsc.store_scatter(mask_v, [plsc.load_gather(y_v, [rowi, cy])], tag)
        if issues is not None:
            _issue_y(issues)
        for j in range(n_sh + 1):
            if j < n_sh:
                q = sh_v[r, pl.ds(j * 16, 16)]
                o = o_v[r, pl.ds(j * 16, 16)]
            else:
                q = jnp.where(m_sh, plsc.load_gather(sh_v, [rowi, csh]),
                              padq)
                o = plsc.load_gather(o_v, [rowi, csh])
            m = plsc.load_gather(mask_v, [q])
            s1 = s1 + jnp.where(m == tag, o, zf)
        if issues is not None:
            _issue_o(issues)
        rowcnt = zi
        for j in range(n_tk + 1):
            if j < n_tk:
                q = tk_v[r, pl.ds(j * 16, 16)]
                spv = sp_v[r, pl.ds(j * 16, 16)]
            else:
                q = jnp.where(m_tk, plsc.load_gather(tk_v, [rowi, ctk]),
                              padq)
                spv = plsc.load_gather(sp_v, [rowi, ctk])
            m = plsc.load_gather(mask_v, [q])
            hit = m == tag
            h = h + jnp.where(hit, spv, zf)
            rowcnt = rowcnt + jnp.where(hit, onei, zi)
        cmax = jnp.maximum(cmax, jnp.sum(rowcnt))
        csum = csum + rowcnt
        return (s1, h, csum, cmax)

    def _compute(ci, slot, carry):
        base = row0 + ci * _CH
        carry = lax.fori_loop(
            0, _CH - 1, lambda r, c: _row(r, base, slot, c, None), carry)
        return _row(_CH - 1, base, slot, carry,
                    jnp.minimum(ci + 1, last))

    def pair_body(p, carry):
        c0 = p * 2
        c1 = c0 + 1
        _wait(c0, 0)
        _wait_y(c0)
        _wait_o(c0)
        carry = _compute(c0, 0, carry)
        _issue(jnp.minimum(c0 + 2, last), 0)
        _wait(c1, 1)
        _wait_y(c1)
        _wait_o(c1)
        carry = _compute(c1, 1, carry)
        _issue(jnp.minimum(c1 + 2, last), 1)
        return carry

    s1, h, csum, cmax = lax.fori_loop(
        0, _NCHUNK // 2, pair_body,
        (zf, zf, zi, jnp.int32(0)))
    _wait(last, 0)
    _wait(last, 1)
    _wait_y(last)
    _wait_o(last)

    nwl = _NW * _LANES
    r1_v[...] = s1
    r2_v[...] = h
    r3_v[...] = csum.astype(jnp.float32)
    r4_v[...] = jnp.full((_LANES,), cmax, jnp.int32).astype(jnp.float32)
    pltpu.sync_copy(r1_v, res_out.at[pl.ds(wid * _LANES, _LANES)])
    pltpu.sync_copy(r2_v, res_out.at[pl.ds(nwl + wid * _LANES, _LANES)])
    pltpu.sync_copy(r3_v, res_out.at[pl.ds(2 * nwl + wid * _LANES, _LANES)])
    pltpu.sync_copy(r4_v, res_out.at[pl.ds(3 * nwl + wid * _LANES, _LANES)])


def _sc_membership(sh, y, tk, out, sp):
    mesh = plsc.VectorSubcoreMesh(core_axis_name="c", subcore_axis_name="s")
    buf_pair = [
        pltpu.VMEM((_CH, _S), jnp.int32),
        pltpu.VMEM((_CH, _K), jnp.int32),
        pltpu.VMEM((_CH, _K), jnp.float32),
    ]
    f = pl.kernel(
        _sc_body,
        out_type=jax.ShapeDtypeStruct((4 * _NW * _LANES,), jnp.float32),
        mesh=mesh,
        compiler_params=pltpu.CompilerParams(
            needs_layout_passes=False, use_tc_tiling_on_sc=True),
        scratch_types=(
            [pltpu.VMEM((_MASKN,), jnp.int32)]
            + buf_pair + buf_pair
            + [
                pltpu.VMEM((_CH, _LY), jnp.int32),
                pltpu.VMEM((_CH, _S), jnp.float32),
                pltpu.VMEM((_LANES,), jnp.float32),
                pltpu.VMEM((_LANES,), jnp.float32),
                pltpu.VMEM((_LANES,), jnp.float32),
                pltpu.VMEM((_LANES,), jnp.float32),
                pltpu.SemaphoreType.DMA,
                pltpu.SemaphoreType.DMA,
                pltpu.SemaphoreType.DMA,
                pltpu.SemaphoreType.DMA,
            ]
        ),
    )
    return f(sh, y, tk, out, sp)


def kernel(out, shorty, topk_C_vals, topk_C_inds, y_inds):
    B, S = out.shape
    sp = _tc_sp(topk_C_vals.T)
    res = _sc_membership(shorty.astype(jnp.int32), y_inds.astype(jnp.int32),
                         topk_C_inds.astype(jnp.int32), out, sp)
    a_arr = _tc_a(out)

    nwl = _NW * _LANES
    A = a_arr[0, 0]
    S1 = jnp.sum(res[:nwl])
    H = jnp.sum(res[nwl:2 * nwl])
    csum = jnp.sum(res[2 * nwl:3 * nwl])
    k = jnp.maximum(jnp.max(res[3 * nwl:]), 1.0)
    loss_precision = (A - S1) / jnp.float32(B * S)
    n = jnp.float32(B) * k
    loss_recall = (H + (n - csum) * jnp.float32(LOG2_)) / n
    return loss_precision + jnp.float32(GAMMA_) * loss_recall


# speedup vs baseline: 1.0427x; 1.0008x over previous
import jax
import jax.numpy as jnp
from jax import lax
from jax.experimental import pallas as pl
from jax.experimental.pallas import tpu as pltpu
from jax.experimental.pallas import tpu_sc as plsc

GAMMA_ = 0.05
LOG2_ = 0.6931471805599453

_NC, _NS, _LANES = 2, 16, 16
_NW = _NC * _NS

_B, _S, _K, _LY = 4096, 500, 200, 50
_PADQ = 100001
_MASKN = 100096

_RPW = _B // _NW
_CH = 8
_NCHUNK = _RPW // _CH


def _tc_sp_body(vals_ref, sp_ref):
    v = vals_ref[...]
    sp = jnp.maximum(v, 0.0) - v + jnp.log(1.0 + jnp.exp(-jnp.abs(v)))
    sp_ref[...] = sp.T


def _tc_sp(vals_t):
    blk = 512
    return pl.pallas_call(
        _tc_sp_body,
        grid=(_B // blk,),
        in_specs=[pl.BlockSpec((_K, blk), lambda i: (0, i))],
        out_specs=pl.BlockSpec((blk, _K), lambda i: (i, 0)),
        out_shape=jax.ShapeDtypeStruct((_B, _K), jnp.float32),
    )(vals_t)


def _tc_a_body(out_ref, a_ref):
    step = pl.program_id(0)
    x = out_ref[...]
    a_part = jnp.sum(jnp.maximum(x, 0.0) + jnp.log(1.0 + jnp.exp(-jnp.abs(x))))

    @pl.when(step == 0)
    def _():
        a_ref[0, 0] = 0.0

    a_ref[0, 0] += a_part


def _tc_a(out):
    blk = 512
    return pl.pallas_call(
        _tc_a_body,
        grid=(_B // blk,),
        in_specs=[pl.BlockSpec((blk, _S), lambda i: (i, 0))],
        out_specs=pl.BlockSpec(memory_space=pltpu.SMEM),
        out_shape=jax.ShapeDtypeStruct((1, 1), jnp.float32),
    )(out)


def _sc_body(sh_hbm, y_hbm, tk_hbm, o_hbm, sp_hbm,
             res_out,
             mask_v, sh_v0, tk_v0, sp_v0,
             sh_v1, tk_v1, sp_v1,
             y_v, o_v,
             r1_v, r2_v, r3_v, r4_v, sem0, sem1, sem_y, sem_o):
    wid = lax.axis_index("s") * _NC + lax.axis_index("c")
    row0 = wid * _RPW
    neg1 = jnp.full((_LANES,), -1, jnp.int32)
    bufs = ((sh_v0, tk_v0, sp_v0), (sh_v1, tk_v1, sp_v1))
    sems = (sem0, sem1)

    def _copies(ci, slot):
        base = row0 + ci * _CH
        sh_v, tk_v, sp_v = bufs[slot]
        return (
            (sh_hbm.at[pl.ds(base, _CH)], sh_v),
            (tk_hbm.at[pl.ds(base, _CH)], tk_v),
            (sp_hbm.at[pl.ds(base, _CH)], sp_v),
        )

    def _issue(ci, slot):
        for src, dst in _copies(ci, slot):
            pltpu.async_copy(src, dst, sems[slot])

    def _wait(ci, slot):
        for src, dst in _copies(ci, slot):
            pltpu.make_async_copy(src, dst, sems[slot]).wait()

    def _issue_y(ci):
        base = row0 + ci * _CH
        pltpu.async_copy(y_hbm.at[pl.ds(base, _CH)], y_v, sem_y)

    def _wait_y(ci):
        base = row0 + ci * _CH
        pltpu.make_async_copy(y_hbm.at[pl.ds(base, _CH)], y_v, sem_y).wait()

    def _issue_o(ci):
        base = row0 + ci * _CH
        pltpu.async_copy(o_hbm.at[pl.ds(base, _CH)], o_v, sem_o)

    def _wait_o(ci):
        base = row0 + ci * _CH
        pltpu.make_async_copy(o_hbm.at[pl.ds(base, _CH)], o_v, sem_o).wait()

    _issue(0, 0)
    _issue(1, 1)
    _issue_y(0)
    _issue_o(0)

    def init_body(i, _):
        for t in range(16):
            mask_v[pl.ds(i * 256 + t * 16, 16)] = neg1
        return 0

    lax.fori_loop(0, _MASKN // 256, init_body, 0)

    zf = jnp.zeros((_LANES,), jnp.float32)
    zi = jnp.zeros((_LANES,), jnp.int32)
    onei = jnp.full((_LANES,), 1, jnp.int32)
    iota = lax.iota(jnp.int32, _LANES)
    padq = jnp.full((_LANES,), _PADQ, jnp.int32)
    m_sh = iota < (_S % 16)
    m_tk = iota < (_K % 16)
    cy = jnp.minimum(iota + (_LY // 16) * 16, _LY - 1)
    csh = jnp.minimum(iota + (_S // 16) * 16, _S - 1)
    ctk = jnp.minimum(iota + (_K // 16) * 16, _K - 1)
    n_y, n_sh, n_tk = _LY // 16, _S // 16, _K // 16
    last = _NCHUNK - 1

    def _row(r, base, slot, carry, issues):
        s1, h, csum, cmax = carry
        sh_v, tk_v, sp_v = bufs[slot]
        tag = jnp.full((_LANES,), base + r, jnp.int32)
        rowi = jnp.full((_LANES,), r, jnp.int32)
        for t in range(n_y):
            plsc.store_scatter(mask_v, [y_v[r, pl.ds(t * 16, 16)]], tag)
        plsc.store_scatter(mask_v, [plsc.load_gather(y_v, [rowi, cy])], tag)
        if issues is not None:
            _issue_y(issues)
        for j in range(n_sh + 1):
            if j < n_sh:
                q = sh_v[r, pl.ds(j * 16, 16)]
                o = o_v[r, pl.ds(j * 16, 16)]
            else:
                q = jnp.where(m_sh, plsc.load_gather(sh_v, [rowi, csh]),
                              padq)
                o = plsc.load_gather(o_v, [rowi, csh])
            m = plsc.load_gather(mask_v, [q])
            s1 = s1 + jnp.where(m == tag, o, zf)
        if issues is not None:
            _issue_o(issues)
        rowcnt = zi
        for j in range(n_tk + 1):
            if j < n_tk:
                q = tk_v[r, pl.ds(j * 16, 16)]
                spv = sp_v[r, pl.ds(j * 16, 16)]
            else:
                q = jnp.where(m_tk, plsc.load_gather(tk_v, [rowi, ctk]),
                              padq)
                spv = plsc.load_gather(sp_v, [rowi, ctk])
            m = plsc.load_gather(mask_v, [q])
            hit = m == tag
            h = h + jnp.where(hit, spv, zf)
            rowcnt = rowcnt + jnp.where(hit, onei, zi)
        cmax = jnp.maximum(cmax, jnp.sum(rowcnt))
        csum = csum + rowcnt
        return (s1, h, csum, cmax)

    def _compute(ci, slot, carry):
        base = row0 + ci * _CH
        carry = lax.fori_loop(
            0, _CH - 1, lambda r, c: _row(r, base, slot, c, None), carry)
        return _row(_CH - 1, base, slot, carry,
                    jnp.minimum(ci + 1, last))

    def pair_body(p, carry):
        c0 = p * 2
        c1 = c0 + 1
        _wait(c0, 0)
        _wait_y(c0)
        _wait_o(c0)
        carry = _compute(c0, 0, carry)
        _issue(jnp.minimum(c0 + 2, last), 0)
        _wait(c1, 1)
        _wait_y(c1)
        _wait_o(c1)
        carry = _compute(c1, 1, carry)
        _issue(jnp.minimum(c1 + 2, last), 1)
        return carry

    s1, h, csum, cmax = lax.fori_loop(
        0, _NCHUNK // 2, pair_body,
        (zf, zf, zi, jnp.int32(0)))
    _wait(last, 0)
    _wait(last, 1)
    _wait_y(last)
    _wait_o(last)

    nwl = _NW * _LANES
    r1_v[...] = s1
    r2_v[...] = h
    r3_v[...] = csum.astype(jnp.float32)
    r4_v[...] = jnp.full((_LANES,), cmax, jnp.int32).astype(jnp.float32)
    pltpu.sync_copy(r1_v, res_out.at[pl.ds(wid * _LANES, _LANES)])
    pltpu.sync_copy(r2_v, res_out.at[pl.ds(nwl + wid * _LANES, _LANES)])
    pltpu.sync_copy(r3_v, res_out.at[pl.ds(2 * nwl + wid * _LANES, _LANES)])
    pltpu.sync_copy(r4_v, res_out.at[pl.ds(3 * nwl + wid * _LANES, _LANES)])


def _sc_membership(sh, y, tk, out, sp):
    mesh = plsc.VectorSubcoreMesh(core_axis_name="c", subcore_axis_name="s")
    buf_pair = [
        pltpu.VMEM((_CH, _S), jnp.int32),
        pltpu.VMEM((_CH, _K), jnp.int32),
        pltpu.VMEM((_CH, _K), jnp.float32),
    ]
    f = pl.kernel(
        _sc_body,
        out_type=jax.ShapeDtypeStruct((4 * _NW * _LANES,), jnp.float32),
        mesh=mesh,
        compiler_params=pltpu.CompilerParams(
            needs_layout_passes=False, use_tc_tiling_on_sc=True),
        scratch_types=(
            [pltpu.VMEM((_MASKN,), jnp.int32)]
            + buf_pair + buf_pair
            + [
                pltpu.VMEM((_CH, _LY), jnp.int32),
                pltpu.VMEM((_CH, _S), jnp.float32),
                pltpu.VMEM((_LANES,), jnp.float32),
                pltpu.VMEM((_LANES,), jnp.float32),
                pltpu.VMEM((_LANES,), jnp.float32),
                pltpu.VMEM((_LANES,), jnp.float32),
                pltpu.SemaphoreType.DMA,
                pltpu.SemaphoreType.DMA,
                pltpu.SemaphoreType.DMA,
                pltpu.SemaphoreType.DMA,
            ]
        ),
    )
    return f(sh, y, tk, out, sp)


def kernel(out, shorty, topk_C_vals, topk_C_inds, y_inds):
    B, S = out.shape
    sp = _tc_sp(topk_C_vals.T)
    res = _sc_membership(shorty.astype(jnp.int32), y_inds.astype(jnp.int32),
                         topk_C_inds.astype(jnp.int32), out, sp)
    a_arr = _tc_a(out)

    nwl = _NW * _LANES
    A = a_arr[0, 0]
    S1 = jnp.sum(res[:nwl])
    H = jnp.sum(res[nwl:2 * nwl])
    csum = jnp.sum(res[2 * nwl:3 * nwl])
    k = jnp.maximum(jnp.max(res[3 * nwl:]), 1.0)
    loss_precision = (A - S1) / jnp.float32(B * S)
    n = jnp.float32(B) * k
    loss_recall = (H + (n - csum) * jnp.float32(LOG2_)) / n
    return loss_precision + jnp.float32(GAMMA_) * loss_recall
